# baseline (speedup 1.0000x reference)
import functools

import jax
import jax.numpy as jnp
from jax import lax
from jax.experimental import pallas as pl
from jax.experimental.pallas import tpu as pltpu
from jax.experimental.pallas import tpu_sc as plsc

_L = 128
_D = 32


@functools.cache
def _build(b, h):
    info = plsc.get_sparse_core_info()
    nw = info.num_cores * info.num_subcores
    nbb = b // _L
    n_units = h * nbb
    upw = n_units // nw

    mesh = plsc.VectorSubcoreMesh(core_axis_name="c", subcore_axis_name="s")

    @functools.partial(
        pl.kernel,
        out_type=jax.ShapeDtypeStruct((h, _D // 8, nbb, 8, _L), jnp.float32),
        mesh=mesh,
        compiler_params=pltpu.CompilerParams(
            use_tc_tiling_on_sc=False, needs_layout_passes=False),
        scratch_types=[
            pltpu.VMEM((upw, _L), jnp.int32),
            pltpu.VMEM((_L, _D), jnp.float32),
            pltpu.VMEM((_L, _D), jnp.float32),
            pltpu.VMEM((_D // 8, 8, _L), jnp.float32),
            pltpu.VMEM((_D // 8, 8, _L), jnp.float32),
            pltpu.SemaphoreType.DMA,
            pltpu.SemaphoreType.DMA,
            pltpu.SemaphoreType.DMA,
            pltpu.SemaphoreType.DMA,
        ],
    )
    def emb(x_hbm, w_hbm, out_hbm, idx_v, rows0, rows1,
            tb0, tb1, gsem0, gsem1, osem0, osem1):
        wid = lax.axis_index("s") * info.num_cores + lax.axis_index("c")
        u0 = wid * upw
        pltpu.sync_copy(x_hbm.at[pl.ds(u0, upw)], idx_v)

        def fire(j, rows, sem):
            pltpu.async_copy(w_hbm.at[idx_v.at[j]], rows, sem)

        def drain(rows, sem):
            pltpu.make_async_copy(w_hbm.at[pl.ds(0, _L)], rows, sem).wait()

        def transpose(rows, tb):
            for eg in range(_D // 8):
                for s in range(8):
                    e = eg * 8 + s
                    col = jnp.full((16,), e, jnp.int32)
                    for lb in range(_L // 16):
                        rix = lb * 16 + lax.iota(jnp.int32, 16)
                        v = plsc.load_gather(rows, [rix, col])
                        tb[eg, s, pl.ds(lb * 16, 16)] = v

        def outfire(j, tb, sem):
            u = u0 + j
            hh = u // nbb
            bb = u % nbb
            pltpu.async_copy(tb, out_hbm.at[hh, :, bb], sem)

        def owait(tb, sem):
            pltpu.make_async_copy(tb, out_hbm.at[0, :, 0], sem).wait()

        def halfstep(j, rows, tb, gsem, osem, rows_nxt, gsem_nxt,
                     prime, last):
            if not last:
                fire(j + 1, rows_nxt, gsem_nxt)
            drain(rows, gsem)
            if not prime:
                owait(tb, osem)
            transpose(rows, tb)
            outfire(j, tb, osem)

        fire(0, rows0, gsem0)
        halfstep(0, rows0, tb0, gsem0, osem0, rows1, gsem1, True, False)
        halfstep(1, rows1, tb1, gsem1, osem1, rows0, gsem0, True, False)

        def body(u, carry):
            a = 2 * u
            halfstep(a, rows0, tb0, gsem0, osem0, rows1, gsem1, False, False)
            halfstep(a + 1, rows1, tb1, gsem1, osem1, rows0, gsem0,
                     False, False)
            return carry

        lax.fori_loop(1, upw // 2 - 1, body, 0)

        a = upw - 2
        halfstep(a, rows0, tb0, gsem0, osem0, rows1, gsem1, False, False)
        halfstep(a + 1, rows1, tb1, gsem1, osem1, rows0, gsem0, False, True)
        owait(tb0, osem0)
        owait(tb1, osem1)

    return emb


def kernel(x, weight):
    b, h = x.shape
    _, d = weight.shape
    xr = x.T.reshape(h * (b // _L), _L).astype(jnp.int32)
    o5 = _build(b, h)(xr, weight)
    return o5.transpose(2, 4, 0, 1, 3).reshape(b, h, d)


# speedup vs baseline: 1.6630x; 1.6630x over previous
import functools

import jax
import jax.numpy as jnp
import numpy as np
from jax import lax
from jax.experimental import pallas as pl
from jax.experimental.pallas import tpu as pltpu
from jax.experimental.pallas import tpu_sc as plsc

_L = 128
_D = 32


@functools.cache
def _build(b, h):
    info = plsc.get_sparse_core_info()
    nw = info.num_cores * info.num_subcores
    nbb = b // _L
    n_units = h * nbb
    upw = n_units // nw

    mesh = plsc.VectorSubcoreMesh(core_axis_name="c", subcore_axis_name="s")

    @functools.partial(
        pl.kernel,
        out_type=jax.ShapeDtypeStruct((h, _D // 8, nbb, 8, _L), jnp.float32),
        mesh=mesh,
        compiler_params=pltpu.CompilerParams(
            use_tc_tiling_on_sc=False, needs_layout_passes=False),
        scratch_types=[
            pltpu.VMEM((upw, _L), jnp.int32),
            pltpu.VMEM((_L, _D), jnp.float32),
            pltpu.VMEM((_L, _D), jnp.float32),
            pltpu.VMEM((_D // 8, 8, _L + 1), jnp.float32),
            pltpu.VMEM((_D // 8, 8, _L + 1), jnp.float32),
            pltpu.SemaphoreType.DMA,
            pltpu.SemaphoreType.DMA,
            pltpu.SemaphoreType.DMA,
            pltpu.SemaphoreType.DMA,
        ],
    )
    def emb(x_hbm, w_hbm, out_hbm, idx_v, rows0, rows1,
            tb0, tb1, gsem0, gsem1, osem0, osem1):
        wid = lax.axis_index("s") * info.num_cores + lax.axis_index("c")
        u0 = wid * upw
        pltpu.sync_copy(x_hbm.at[pl.ds(u0, upw)], idx_v)

        def fire(j, rows, sem):
            pltpu.async_copy(w_hbm.at[idx_v.at[j]], rows, sem)

        def drain(rows, sem):
            pltpu.make_async_copy(w_hbm.at[pl.ds(0, _L)], rows, sem).wait()

        ar = lax.iota(jnp.int32, 16)
        eg0 = lax.shift_right_logical(ar, 3)
        eg1 = eg0 + 2
        s_v = lax.bitwise_and(ar, 7)

        def transpose(rows, tb):
            for l in range(_L):
                lv = jnp.full((16,), l, jnp.int32)
                plsc.store_scatter(tb, [eg0, s_v, lv], rows[l, pl.ds(0, 16)])
                plsc.store_scatter(tb, [eg1, s_v, lv], rows[l, pl.ds(16, 16)])

        def outfire(j, tb, sem):
            u = u0 + j
            hh = u // nbb
            bb = u % nbb
            pltpu.async_copy(tb.at[:, :, pl.ds(0, _L)], out_hbm.at[hh, :, bb],
                             sem)

        def owait(tb, sem):
            pltpu.make_async_copy(tb.at[:, :, pl.ds(0, _L)],
                                  out_hbm.at[0, :, 0], sem).wait()

        def halfstep(j, rows, tb, gsem, osem, rows_nxt, gsem_nxt,
                     prime, last):
            if not last:
                fire(j + 1, rows_nxt, gsem_nxt)
            drain(rows, gsem)
            if not prime:
                owait(tb, osem)
            transpose(rows, tb)
            outfire(j, tb, osem)

        fire(0, rows0, gsem0)
        halfstep(0, rows0, tb0, gsem0, osem0, rows1, gsem1, True, False)
        halfstep(1, rows1, tb1, gsem1, osem1, rows0, gsem0, True, False)

        def body(u, carry):
            a = 2 * u
            halfstep(a, rows0, tb0, gsem0, osem0, rows1, gsem1, False, False)
            halfstep(a + 1, rows1, tb1, gsem1, osem1, rows0, gsem0,
                     False, False)
            return carry

        lax.fori_loop(1, upw // 2 - 1, body, 0)

        a = upw - 2
        halfstep(a, rows0, tb0, gsem0, osem0, rows1, gsem1, False, False)
        halfstep(a + 1, rows1, tb1, gsem1, osem1, rows0, gsem0, False, True)
        owait(tb0, osem0)
        owait(tb1, osem1)

    return emb


def kernel(x, weight):
    b, h = x.shape
    _, d = weight.shape
    xr = x.T.reshape(h * (b // _L), _L).astype(jnp.int32)
    o5 = _build(b, h)(xr, weight)
    return o5.transpose(2, 4, 0, 1, 3).reshape(b, h, d)
